# baseline (speedup 1.0000x reference)
import functools

import jax
import jax.numpy as jnp
from jax import lax
from jax.experimental import pallas as pl
from jax.experimental.pallas import tpu as pltpu
from jax.experimental.pallas import tpu_sc as plsc

B, C, H, W = 4, 384, 224, 224
HW = H * W
NPLANES = B * C
PAD = HW

NC, NS, L = 2, 16, 16
NW = NC * NS
PLANES_PER_W = NPLANES // NW
P = 3584
NCHUNK = HW // P
NVEC = P // L


def _precompute_body(d_ref, pk1_ref, pk2_ref, fx_ref, fy_ref):
    i = pl.program_id(1)
    rows = d_ref.shape[2]
    hh = (
        lax.broadcasted_iota(jnp.int32, (rows, W), 0) + i * rows
    ).astype(jnp.float32)
    ww = lax.broadcasted_iota(jnp.int32, (rows, W), 1).astype(jnp.float32)
    dy = d_ref[0, 0]
    dx = d_ref[0, 1]
    gy = hh + dy
    gx = ww + dx
    ny = 2.0 * (gy / (H - 1) - 0.5)
    nx = 2.0 * (gx / (W - 1) - 0.5)
    y = (ny + 1.0) * 0.5 * (H - 1)
    x = (nx + 1.0) * 0.5 * (W - 1)
    x0f = jnp.floor(x)
    y0f = jnp.floor(y)
    x1f = x0f + 1.0
    y1f = y0f + 1.0
    fx_ref[0] = x - x0f
    fy_ref[0] = y - y0f
    inx0 = (x0f >= 0.0) & (x0f <= W - 1.0)
    inx1 = (x1f >= 0.0) & (x1f <= W - 1.0)
    iny0 = (y0f >= 0.0) & (y0f <= H - 1.0)
    iny1 = (y1f >= 0.0) & (y1f <= H - 1.0)
    x0c = jnp.clip(x0f, 0.0, W - 1.0).astype(jnp.int32)
    x1c = jnp.clip(x1f, 0.0, W - 1.0).astype(jnp.int32)
    y0r = jnp.clip(y0f, 0.0, H - 1.0).astype(jnp.int32) * W
    y1r = jnp.clip(y1f, 0.0, H - 1.0).astype(jnp.int32) * W
    pad = jnp.int32(PAD)
    ia = jnp.where(inx0 & iny0, y0r + x0c, pad)
    ib = jnp.where(inx0 & iny1, y1r + x0c, pad)
    ic = jnp.where(inx1 & iny0, y0r + x1c, pad)
    id_ = jnp.where(inx1 & iny1, y1r + x1c, pad)
    pk1_ref[0] = ia | (ib << 16)
    pk2_ref[0] = ic | (id_ << 16)


def _precompute(deformation_field):
    rows = 16
    grid = (B, H // rows)
    out_shape = [
        jax.ShapeDtypeStruct((B, H, W), jnp.int32),
        jax.ShapeDtypeStruct((B, H, W), jnp.int32),
        jax.ShapeDtypeStruct((B, H, W), jnp.float32),
        jax.ShapeDtypeStruct((B, H, W), jnp.float32),
    ]
    ospec = pl.BlockSpec((1, rows, W), lambda b, i: (b, i, 0))
    return pl.pallas_call(
        _precompute_body,
        grid=grid,
        in_specs=[pl.BlockSpec((1, 2, rows, W), lambda b, i: (b, 0, i, 0))],
        out_specs=[ospec, ospec, ospec, ospec],
        out_shape=out_shape,
    )(deformation_field)


def _sc_body(f2d, pk1, pk2, fxa, fya, out, plane_v, pk1_v, pk2_v, fx_v, fy_v, out_v):
    wid = lax.axis_index("s") * NC + lax.axis_index("c")
    base = (wid // (NW // B)) * HW
    plane0 = wid * PLANES_PER_W
    plane_v[pl.ds(HW, L)] = jnp.zeros((L,), jnp.float32)

    def plane_loop(p, _):
        plane = plane0 + p
        pltpu.sync_copy(f2d.at[plane], plane_v.at[pl.ds(0, HW)])

        def chunk_loop(jc, _):
            off = base + jc * P
            pltpu.sync_copy(pk1.at[pl.ds(off, P)], pk1_v)
            pltpu.sync_copy(pk2.at[pl.ds(off, P)], pk2_v)
            pltpu.sync_copy(fxa.at[pl.ds(off, P)], fx_v)
            pltpu.sync_copy(fya.at[pl.ds(off, P)], fy_v)

            def inner(i, _):
                sl = pl.ds(i * L, L)
                p1 = pk1_v[sl]
                p2 = pk2_v[sl]
                fx = fx_v[sl]
                fy = fy_v[sl]
                mask = jnp.full((L,), 0xFFFF, jnp.int32)
                ia = p1 & mask
                ib = lax.shift_right_logical(p1, 16)
                ic = p2 & mask
                id_ = lax.shift_right_logical(p2, 16)
                ga = plsc.load_gather(plane_v, [ia])
                gb = plsc.load_gather(plane_v, [ib])
                gc = plsc.load_gather(plane_v, [ic])
                gd = plsc.load_gather(plane_v, [id_])
                ax = 1.0 - fx
                ay = 1.0 - fy
                r = ga * (ax * ay) + gb * (ax * fy) + gc * (fx * ay) + gd * (fx * fy)
                out_v[sl] = r
                return _

            lax.fori_loop(0, NVEC, inner, None, unroll=8)
            pltpu.sync_copy(out_v, out.at[plane, pl.ds(jc * P, P)])
            return _

        lax.fori_loop(0, NCHUNK, chunk_loop, None)
        return _

    lax.fori_loop(0, PLANES_PER_W, plane_loop, None)


@jax.jit
def _sc_gather(f2d, pk1, pk2, fxa, fya):
    mesh = plsc.VectorSubcoreMesh(
        core_axis_name="c", subcore_axis_name="s", num_cores=NC, num_subcores=NS
    )
    return pl.kernel(
        _sc_body,
        out_type=jax.ShapeDtypeStruct((NPLANES, HW), jnp.float32),
        mesh=mesh,
        compiler_params=pltpu.CompilerParams(needs_layout_passes=False),
        scratch_types=[
            pltpu.VMEM((HW + L,), jnp.float32),
            pltpu.VMEM((P,), jnp.int32),
            pltpu.VMEM((P,), jnp.int32),
            pltpu.VMEM((P,), jnp.float32),
            pltpu.VMEM((P,), jnp.float32),
            pltpu.VMEM((P,), jnp.float32),
        ],
    )(f2d, pk1, pk2, fxa, fya)


def kernel(f_pri, deformation_field):
    pk1, pk2, fx, fy = _precompute(deformation_field)
    f2d = f_pri.reshape(NPLANES, HW)
    out2d = _sc_gather(
        f2d,
        pk1.reshape(B * HW),
        pk2.reshape(B * HW),
        fx.reshape(B * HW),
        fy.reshape(B * HW),
    )
    return out2d.reshape(B, C, H, W)


# speedup vs baseline: 2.2524x; 2.2524x over previous
import jax
import jax.numpy as jnp
from jax import lax
from jax.experimental import pallas as pl
from jax.experimental.pallas import tpu as pltpu
from jax.experimental.pallas import tpu_sc as plsc

B, C, H, W = 4, 384, 224, 224
HW = H * W
NPLANES = B * C
PAD = HW

NC, NS, L = 2, 16, 16
NW = NC * NS
PLANES_PER_W = NPLANES // NW

ROWS = 16
P = ROWS * W
NCHUNK = HW // P
IP = 4 * P


def _precompute_body(d_ref, iw_ref):
    i = pl.program_id(1)
    hh = (lax.broadcasted_iota(jnp.int32, (ROWS, W), 0) + i * ROWS).astype(
        jnp.float32
    )
    ww = lax.broadcasted_iota(jnp.int32, (ROWS, W), 1).astype(jnp.float32)
    dy = d_ref[0, 0]
    dx = d_ref[0, 1]
    gy = hh + dy
    gx = ww + dx
    ny = 2.0 * (gy / (H - 1) - 0.5)
    nx = 2.0 * (gx / (W - 1) - 0.5)
    y = (ny + 1.0) * 0.5 * (H - 1)
    x = (nx + 1.0) * 0.5 * (W - 1)
    x0f = jnp.floor(x)
    y0f = jnp.floor(y)
    x1f = x0f + 1.0
    y1f = y0f + 1.0
    inx0 = (x0f >= 0.0) & (x0f <= W - 1.0)
    inx1 = (x1f >= 0.0) & (x1f <= W - 1.0)
    iny0 = (y0f >= 0.0) & (y0f <= H - 1.0)
    iny1 = (y1f >= 0.0) & (y1f <= H - 1.0)
    x0c = jnp.clip(x0f, 0.0, W - 1.0).astype(jnp.int32)
    x1c = jnp.clip(x1f, 0.0, W - 1.0).astype(jnp.int32)
    y0r = jnp.clip(y0f, 0.0, H - 1.0).astype(jnp.int32) * W
    y1r = jnp.clip(y1f, 0.0, H - 1.0).astype(jnp.int32) * W
    pad = jnp.int32(PAD)
    ia = jnp.where(inx0 & iny0, y0r + x0c, pad)
    ib = jnp.where(inx0 & iny1, y1r + x0c, pad)
    ic = jnp.where(inx1 & iny0, y0r + x1c, pad)
    id_ = jnp.where(inx1 & iny1, y1r + x1c, pad)
    iw_ref[0, 0, 0] = lax.bitcast_convert_type(ia | (ib << 16), jnp.float32)
    iw_ref[0, 0, 1] = lax.bitcast_convert_type(ic | (id_ << 16), jnp.float32)
    iw_ref[0, 0, 2] = x - x0f
    iw_ref[0, 0, 3] = y - y0f


def _precompute(deformation_field):
    return pl.pallas_call(
        _precompute_body,
        grid=(B, NCHUNK),
        in_specs=[pl.BlockSpec((1, 2, ROWS, W), lambda b, i: (b, 0, i, 0))],
        out_specs=pl.BlockSpec((1, 1, 4, ROWS, W), lambda b, i: (b, i, 0, 0, 0)),
        out_shape=jax.ShapeDtypeStruct((B, NCHUNK, 4, ROWS, W), jnp.float32),
    )(deformation_field)


def _sc_body(f2d, idxw, out, plane_v, ibuf, obuf, in_sems, out_sems, plane_sem):
    wid = lax.axis_index("s") * NC + lax.axis_index("c")
    b = wid // (NW // B)
    ibase = b * NCHUNK * IP
    plane0 = wid * PLANES_PER_W
    plane_v[pl.ds(HW, L)] = jnp.zeros((L,), jnp.float32)

    def start_in(jc, slot):
        return pltpu.async_copy(
            idxw.at[pl.ds(ibase + jc * IP, IP)], ibuf.at[slot], in_sems.at[slot]
        )

    def plane_loop(p, _):
        plane = plane0 + p
        pltpu.async_copy(f2d.at[plane], plane_v.at[pl.ds(0, HW)], plane_sem)
        start_in(0, 0)
        pltpu.make_async_copy(
            f2d.at[plane], plane_v.at[pl.ds(0, HW)], plane_sem
        ).wait()

        def chunk_loop(jc, _):
            slot = jc % 2

            @pl.when(jc + 1 < NCHUNK)
            def _():
                start_in(jc + 1, (jc + 1) % 2)

            pltpu.make_async_copy(
                idxw.at[pl.ds(ibase, IP)], ibuf.at[slot], in_sems.at[slot]
            ).wait()

            @pl.when(jc >= 2)
            def _():
                pltpu.make_async_copy(
                    obuf.at[slot], out.at[plane, pl.ds(0, P)], out_sems.at[slot]
                ).wait()

            @plsc.parallel_loop(0, P, step=L, unroll=4)
            def _(i):
                p1 = plsc.bitcast(ibuf[slot, pl.ds(i, L)], jnp.int32)
                p2 = plsc.bitcast(ibuf[slot, pl.ds(P + i, L)], jnp.int32)
                fx = ibuf[slot, pl.ds(2 * P + i, L)]
                fy = ibuf[slot, pl.ds(3 * P + i, L)]
                mask = jnp.full((L,), 0xFFFF, jnp.int32)
                ia = p1 & mask
                ib = lax.shift_right_logical(p1, 16)
                ic = p2 & mask
                id_ = lax.shift_right_logical(p2, 16)
                ga = plsc.load_gather(plane_v, [ia])
                gb = plsc.load_gather(plane_v, [ib])
                gc = plsc.load_gather(plane_v, [ic])
                gd = plsc.load_gather(plane_v, [id_])
                ax = 1.0 - fx
                ay = 1.0 - fy
                obuf[slot, pl.ds(i, L)] = (
                    ga * (ax * ay)
                    + gb * (ax * fy)
                    + gc * (fx * ay)
                    + gd * (fx * fy)
                )

            pltpu.async_copy(
                obuf.at[slot], out.at[plane, pl.ds(jc * P, P)], out_sems.at[slot]
            )
            return _

        lax.fori_loop(0, NCHUNK, chunk_loop, None)
        pltpu.make_async_copy(
            obuf.at[0], out.at[plane, pl.ds(0, P)], out_sems.at[0]
        ).wait()
        pltpu.make_async_copy(
            obuf.at[1], out.at[plane, pl.ds(0, P)], out_sems.at[1]
        ).wait()
        return _

    lax.fori_loop(0, PLANES_PER_W, plane_loop, None)


@jax.jit
def _sc_gather(f2d, idxw):
    mesh = plsc.VectorSubcoreMesh(
        core_axis_name="c", subcore_axis_name="s", num_cores=NC, num_subcores=NS
    )
    return pl.kernel(
        _sc_body,
        out_type=jax.ShapeDtypeStruct((NPLANES, HW), jnp.float32),
        mesh=mesh,
        compiler_params=pltpu.CompilerParams(needs_layout_passes=False),
        scratch_types=[
            pltpu.VMEM((HW + L,), jnp.float32),
            pltpu.VMEM((2, IP), jnp.float32),
            pltpu.VMEM((2, P), jnp.float32),
            pltpu.SemaphoreType.DMA((2,)),
            pltpu.SemaphoreType.DMA((2,)),
            pltpu.SemaphoreType.DMA,
        ],
    )(f2d, idxw)


def kernel(f_pri, deformation_field):
    idxw = _precompute(deformation_field)
    f2d = f_pri.reshape(NPLANES, HW)
    out2d = _sc_gather(f2d, idxw.reshape(B * NCHUNK * IP))
    return out2d.reshape(B, C, H, W)
